# baseline (speedup 1.0000x reference)
import functools

import jax
import jax.numpy as jnp
from jax import lax
from jax.experimental import pallas as pl
from jax.experimental.pallas import tpu as pltpu
from jax.experimental.pallas import tpu_sc as plsc

N = 10000
M = 5000
NNZ = 320000
DIN = 128
DHID = 128
NCLS = 40

NC = 2
NS = 16
L = 16

NP = 10240
MP = 5120
CHUNK = 128
K = 160
NBUF = 4
NNZ_PT = K * CHUNK


def _dv_scale(cnt):
    return jnp.where(cnt > 0, lax.rsqrt(jnp.maximum(cnt, 1e-12)), 0.0)


def _de_scale(cnt):
    return jnp.where(cnt > 0, 1.0 / jnp.maximum(cnt, 1e-12), 0.0)


def _mm1_body(x_ref, w_ref, b_ref, cnw_ref, cmw_ref, out_ref, dvw_ref, dew_ref):
    dv16 = _dv_scale(cnw_ref[...])
    dvw_ref[...] = dv16
    dew_ref[...] = _de_scale(cmw_ref[...])
    h = jnp.dot(x_ref[...], w_ref[...], preferred_element_type=jnp.float32)
    h = (h + b_ref[...]) * dv16[:, :1]
    out_ref[0] = h[:, :64]
    out_ref[1] = h[:, 64:]


def _mm2_body(n_ref, cnt_ref, w_ref, b_ref, out_ref):
    full = jnp.concatenate([n_ref[0], n_ref[1]], axis=-1)
    dv = _dv_scale(cnt_ref[...])
    h = jnp.maximum(full, 0.0)
    h2 = jnp.dot(h, w_ref[...], preferred_element_type=jnp.float32)
    h2 = (h2 + b_ref[...]) * dv
    out_ref[0] = h2[:, :32]
    out_ref[1] = h2[:, 32:]


def _tc_mm1(Xp, W1, b1r, cNw, cMw):
    B = 1024
    bm = MP // (NP // B)
    return pl.pallas_call(
        _mm1_body,
        grid=(NP // B,),
        in_specs=[
            pl.BlockSpec((B, DIN), lambda i: (i, 0)),
            pl.BlockSpec((DIN, DHID), lambda i: (0, 0)),
            pl.BlockSpec((1, DHID), lambda i: (0, 0)),
            pl.BlockSpec((B, L), lambda i: (i, 0)),
            pl.BlockSpec((bm, L), lambda i: (i, 0)),
        ],
        out_specs=[
            pl.BlockSpec((2, B, 64), lambda i: (0, i, 0)),
            pl.BlockSpec((B, L), lambda i: (i, 0)),
            pl.BlockSpec((bm, L), lambda i: (i, 0)),
        ],
        out_shape=[
            jax.ShapeDtypeStruct((2, NP, 64), jnp.float32),
            jax.ShapeDtypeStruct((NP, L), jnp.float32),
            jax.ShapeDtypeStruct((MP, L), jnp.float32),
        ],
    )(Xp, W1, b1r, cNw, cMw)


def _tc_mm2(n1, cN, W2p, b2r):
    B = 1024
    return pl.pallas_call(
        _mm2_body,
        grid=(NP // B,),
        in_specs=[
            pl.BlockSpec((2, B, 64), lambda i: (0, i, 0)),
            pl.BlockSpec((B, 1), lambda i: (i, 0)),
            pl.BlockSpec((DHID, 64), lambda i: (0, 0)),
            pl.BlockSpec((1, 64), lambda i: (0, 0)),
        ],
        out_specs=pl.BlockSpec((2, B, 32), lambda i: (0, i, 0)),
        out_shape=jax.ShapeDtypeStruct((2, NP, 32), jnp.float32),
    )(n1, cN, W2p, b2r)


def _zero_rows(buf, nrows, dh):
    z = jnp.zeros((L,), jnp.float32)

    def row(r, _):
        for d in range(dh // L):
            buf[r, pl.ds(d * L, L)] = z
        return 0

    lax.fori_loop(0, nrows, row, 0)


def _make_stage(rsrc, rdst, dh, interleave=False):
    rows_pt = rdst // NS
    nbuf = NBUF if rdst == NP else 5
    out_shape = (rdst, 2 * dh) if interleave else (2 * rdst, dh)
    mesh = plsc.VectorSubcoreMesh(
        core_axis_name="c", subcore_axis_name="s",
        num_cores=NC, num_subcores=NS)

    @functools.partial(
        pl.kernel,
        out_type=jax.ShapeDtypeStruct(out_shape, jnp.float32),
        mesh=mesh,
        compiler_params=pltpu.CompilerParams(use_tc_tiling_on_sc=False),
        scratch_types=[
            pltpu.VMEM((K, CHUNK), jnp.int32),
            pltpu.VMEM((K, CHUNK), jnp.int32),
            [pltpu.VMEM((CHUNK, dh), jnp.float32) for _ in range(nbuf)],
            [pltpu.SemaphoreType.DMA for _ in range(nbuf)],
            pltpu.VMEM((rows_pt, L), jnp.float32),
            pltpu.VMEM_SHARED((rdst, dh), jnp.float32),
        ],
    )
    def stage(src, gidx, sidx, scalew, dst, gv, sv, bufs, gsems, scb, acc):
        cid = lax.axis_index("c")
        sid = lax.axis_index("s")
        wid = cid * NS + sid
        pltpu.sync_copy(gidx.at[wid], gv)
        pltpu.sync_copy(sidx.at[sid], sv)

        _zero_rows(bufs[0], 64, dh)
        r0 = sid * rows_pt

        def zcp(kk, _):
            pltpu.sync_copy(bufs[0].at[pl.ds(0, 64)],
                            acc.at[pl.ds(r0 + kk * 64, 64)])
            return 0

        lax.fori_loop(0, rows_pt // 64, zcp, 0)
        pltpu.sync_copy(scalew.at[pl.ds(r0, rows_pt)], scb)
        plsc.subcore_barrier()

        def gfire(j, p):
            pltpu.async_copy(src.at[gv.at[j]], bufs[p], gsems[p])

        def gwait(j, p):
            pltpu.make_async_copy(src.at[gv.at[j]], bufs[p], gsems[p]).wait()

        for p in range(nbuf):
            gfire(p, p)

        def body(j4, _):
            j = j4 * nbuf
            for p in range(nbuf):
                gwait(j + p, p)
                pltpu.sync_copy(bufs[p], acc.at[sv.at[j + p]], add=True)
                gfire(j + p + nbuf, p)
            return 0

        lax.fori_loop(0, K // nbuf - 1, body, 0)
        for p in range(nbuf):
            jlast = K - nbuf + p
            gwait(jlast, p)
            pltpu.sync_copy(bufs[p], acc.at[sv.at[jlast]], add=True)
        plsc.subcore_barrier()

        def cblk(b, _):
            blk0 = r0 + b * 64
            pltpu.sync_copy(acc.at[pl.ds(blk0, 64)], bufs[0].at[pl.ds(0, 64)])

            def srow(r, _):
                sval = scb[b * 64 + r]
                for d in range(dh // L):
                    bufs[0][r, pl.ds(d * L, L)] = (
                        bufs[0][r, pl.ds(d * L, L)] * sval)
                return 0

            lax.fori_loop(0, 64, srow, 0)
            if interleave:
                @pl.when(cid == 0)
                def _():
                    pltpu.sync_copy(bufs[0].at[pl.ds(0, 64)],
                                    dst.at[pl.ds(blk0, 64), pl.ds(0, dh)])

                @pl.when(cid == 1)
                def _():
                    pltpu.sync_copy(bufs[0].at[pl.ds(0, 64)],
                                    dst.at[pl.ds(blk0, 64), pl.ds(dh, dh)])
            else:
                pltpu.sync_copy(bufs[0].at[pl.ds(0, 64)],
                                dst.at[pl.ds(cid * rdst + blk0, 64)])
            return 0

        lax.fori_loop(0, rows_pt // 64, cblk, 0)

    return stage


def _make_degrees():
    mesh = plsc.VectorSubcoreMesh(
        core_axis_name="c", subcore_axis_name="s",
        num_cores=NC, num_subcores=NS)
    n_pt = NP // NS
    m_pt = MP // NS

    @functools.partial(
        pl.kernel,
        out_type=(jax.ShapeDtypeStruct((NP, L), jnp.float32),
                  jax.ShapeDtypeStruct((MP, L), jnp.float32)),
        mesh=mesh,
        compiler_params=pltpu.CompilerParams(use_tc_tiling_on_sc=False),
        scratch_types=[
            pltpu.VMEM((K, CHUNK), jnp.int32),
            pltpu.VMEM((CHUNK, L), jnp.float32),
            pltpu.VMEM_SHARED((NP, L), jnp.float32),
            pltpu.SemaphoreType.DMA,
        ],
    )
    def degrees(sn, se, zerosN, onesH, outN, outM, sv, ones, acc, dsem):
        cid = lax.axis_index("c")
        sid = lax.axis_index("s")
        pltpu.sync_copy(onesH, ones)

        @pl.when(cid == 0)
        def _():
            pltpu.sync_copy(sn.at[sid], sv)
            pltpu.sync_copy(zerosN.at[pl.ds(sid * n_pt, n_pt)],
                            acc.at[pl.ds(sid * n_pt, n_pt)])

        @pl.when(cid == 1)
        def _():
            pltpu.sync_copy(se.at[sid], sv)
            pltpu.sync_copy(zerosN.at[pl.ds(sid * m_pt, m_pt)],
                            acc.at[pl.ds(sid * m_pt, m_pt)])

        plsc.subcore_barrier()

        def body(j, _):
            pltpu.async_copy(ones, acc.at[sv.at[j]], dsem, add=True)
            return 0

        lax.fori_loop(0, K, body, 0)

        def drain(j, _):
            pltpu.make_async_copy(ones, acc.at[sv.at[0]], dsem).wait()
            return 0

        lax.fori_loop(0, K, drain, 0)
        plsc.subcore_barrier()

        @pl.when(cid == 0)
        def _():
            pltpu.sync_copy(acc.at[pl.ds(sid * n_pt, n_pt)],
                            outN.at[pl.ds(sid * n_pt, n_pt)])

        @pl.when(cid == 1)
        def _():
            pltpu.sync_copy(acc.at[pl.ds(sid * m_pt, m_pt)],
                            outM.at[pl.ds(sid * m_pt, m_pt)])

    return degrees


_make_stage = functools.lru_cache(maxsize=None)(_make_stage)
_make_degrees = functools.lru_cache(maxsize=None)(_make_degrees)


def kernel(X, node_idx, edge_idx, W1, b1, W2, b2):
    node_idx = node_idx.astype(jnp.int32)
    edge_idx = edge_idx.astype(jnp.int32)

    pad = NS * NNZ_PT - NNZ
    spread = jnp.arange(pad, dtype=jnp.int32)
    gn = jnp.concatenate([node_idx, spread % N]).reshape(NS, K, CHUNK)
    ge = jnp.concatenate([edge_idx, spread % M]).reshape(NS, K, CHUNK)
    sn = jnp.concatenate([node_idx, N + spread % (NP - N)]).reshape(NS, K, CHUNK)
    se = jnp.concatenate([edge_idx, M + spread % (MP - M)]).reshape(NS, K, CHUNK)
    gn_off = jnp.stack([gn, gn + NP]).reshape(NC * NS, K, CHUNK)
    ge_off = jnp.stack([ge, ge + MP]).reshape(NC * NS, K, CHUNK)

    zerosN = jnp.zeros((NP, L), jnp.float32)
    onesH = jnp.ones((CHUNK, L), jnp.float32)
    cNw, cMw = _make_degrees()(sn, se, zerosN, onesH)

    Xp = jnp.pad(X, ((0, NP - N), (0, 0)))
    h, dvw, dew = _tc_mm1(Xp, W1, b1.reshape(1, DHID), cNw, cMw)

    e1 = _make_stage(NP, MP, 64)(h.reshape(2 * NP, 64), gn_off, se, dew)
    n1 = _make_stage(MP, NP, 64)(e1, ge_off, sn, dvw)

    W2p = jnp.pad(W2, ((0, 0), (0, 64 - NCLS)))
    b2r = jnp.pad(b2, (0, 64 - NCLS)).reshape(1, 64)
    h2 = _tc_mm2(n1.reshape(2, NP, 64), cNw[:, :1], W2p, b2r)

    e2 = _make_stage(NP, MP, 32)(h2.reshape(2 * NP, 32), gn_off, se, dew)
    out = _make_stage(MP, NP, 32, interleave=True)(e2, ge_off, sn, dvw)
    return out[:N, :NCLS]


# speedup vs baseline: 1.0073x; 1.0073x over previous
import functools

import jax
import jax.numpy as jnp
from jax import lax
from jax.experimental import pallas as pl
from jax.experimental.pallas import tpu as pltpu
from jax.experimental.pallas import tpu_sc as plsc

N = 10000
M = 5000
NNZ = 320000
DIN = 128
DHID = 128
NCLS = 40

NC = 2
NS = 16
L = 16

NP = 10240
MP = 5120
CHUNK = 128
K = 160
NBUF = 4
NNZ_PT = K * CHUNK


def _dv_scale(cnt):
    return jnp.where(cnt > 0, lax.rsqrt(jnp.maximum(cnt, 1e-12)), 0.0)


def _de_scale(cnt):
    return jnp.where(cnt > 0, 1.0 / jnp.maximum(cnt, 1e-12), 0.0)


def _mm1_body(x_ref, w_ref, b_ref, cnw_ref, cmw_ref, out_ref, dvw_ref, dew_ref):
    dv16 = _dv_scale(cnw_ref[...])
    dvw_ref[...] = dv16
    dew_ref[...] = _de_scale(cmw_ref[...])
    h = jnp.dot(x_ref[...], w_ref[...], preferred_element_type=jnp.float32)
    h = (h + b_ref[...]) * dv16[:, :1]
    out_ref[0] = h[:, :64]
    out_ref[1] = h[:, 64:]


def _mm2_body(n_ref, cnt_ref, w_ref, b_ref, out_ref):
    full = jnp.concatenate([n_ref[0], n_ref[1]], axis=-1)
    dv = _dv_scale(cnt_ref[...])
    h = jnp.maximum(full, 0.0)
    h2 = jnp.dot(h, w_ref[...], preferred_element_type=jnp.float32)
    h2 = (h2 + b_ref[...]) * dv
    out_ref[0] = h2[:, :32]
    out_ref[1] = h2[:, 32:]


def _tc_mm1(Xp, W1, b1r, cNw, cMw):
    B = 1024
    bm = MP // (NP // B)
    return pl.pallas_call(
        _mm1_body,
        grid=(NP // B,),
        in_specs=[
            pl.BlockSpec((B, DIN), lambda i: (i, 0)),
            pl.BlockSpec((DIN, DHID), lambda i: (0, 0)),
            pl.BlockSpec((1, DHID), lambda i: (0, 0)),
            pl.BlockSpec((B, L), lambda i: (i, 0)),
            pl.BlockSpec((bm, L), lambda i: (i, 0)),
        ],
        out_specs=[
            pl.BlockSpec((2, B, 64), lambda i: (0, i, 0)),
            pl.BlockSpec((B, L), lambda i: (i, 0)),
            pl.BlockSpec((bm, L), lambda i: (i, 0)),
        ],
        out_shape=[
            jax.ShapeDtypeStruct((2, NP, 64), jnp.float32),
            jax.ShapeDtypeStruct((NP, L), jnp.float32),
            jax.ShapeDtypeStruct((MP, L), jnp.float32),
        ],
    )(Xp, W1, b1r, cNw, cMw)


def _tc_mm2(n1, cN, W2p, b2r):
    B = 1024
    return pl.pallas_call(
        _mm2_body,
        grid=(NP // B,),
        in_specs=[
            pl.BlockSpec((2, B, 64), lambda i: (0, i, 0)),
            pl.BlockSpec((B, 1), lambda i: (i, 0)),
            pl.BlockSpec((DHID, 64), lambda i: (0, 0)),
            pl.BlockSpec((1, 64), lambda i: (0, 0)),
        ],
        out_specs=pl.BlockSpec((2, B, 32), lambda i: (0, i, 0)),
        out_shape=jax.ShapeDtypeStruct((2, NP, 32), jnp.float32),
    )(n1, cN, W2p, b2r)


def _zero_rows(buf, nrows, dh):
    z = jnp.zeros((L,), jnp.float32)

    def row(r, _):
        for d in range(dh // L):
            buf[r, pl.ds(d * L, L)] = z
        return 0

    lax.fori_loop(0, nrows, row, 0)


def _make_stage(rsrc, rdst, dh, interleave=False):
    rows_pt = rdst // NS
    nbuf = NBUF if rdst == NP else 5
    out_shape = (rdst, 2 * dh) if interleave else (2 * rdst, dh)
    mesh = plsc.VectorSubcoreMesh(
        core_axis_name="c", subcore_axis_name="s",
        num_cores=NC, num_subcores=NS)

    @functools.partial(
        pl.kernel,
        out_type=jax.ShapeDtypeStruct(out_shape, jnp.float32),
        mesh=mesh,
        compiler_params=pltpu.CompilerParams(use_tc_tiling_on_sc=False),
        scratch_types=[
            pltpu.VMEM((K, CHUNK), jnp.int32),
            pltpu.VMEM((K, CHUNK), jnp.int32),
            [pltpu.VMEM((CHUNK, dh), jnp.float32) for _ in range(nbuf)],
            [pltpu.SemaphoreType.DMA for _ in range(nbuf)],
            pltpu.VMEM((rows_pt, L), jnp.float32),
            pltpu.VMEM_SHARED((rdst, dh), jnp.float32),
        ],
    )
    def stage(src, gidx, sidx, scalew, dst, gv, sv, bufs, gsems, scb, acc):
        cid = lax.axis_index("c")
        sid = lax.axis_index("s")
        wid = cid * NS + sid
        pltpu.sync_copy(gidx.at[wid], gv)
        pltpu.sync_copy(sidx.at[sid], sv)

        _zero_rows(bufs[0], 64, dh)
        r0 = sid * rows_pt

        def zcp(kk, _):
            pltpu.sync_copy(bufs[0].at[pl.ds(0, 64)],
                            acc.at[pl.ds(r0 + kk * 64, 64)])
            return 0

        lax.fori_loop(0, rows_pt // 64, zcp, 0)
        pltpu.sync_copy(scalew.at[pl.ds(r0, rows_pt)], scb)
        plsc.subcore_barrier()

        def gfire(j, p):
            pltpu.async_copy(src.at[gv.at[j]], bufs[p], gsems[p])

        def gwait(j, p):
            pltpu.make_async_copy(src.at[gv.at[j]], bufs[p], gsems[p]).wait()

        for p in range(nbuf):
            gfire(p, p)

        def body(j4, _):
            j = j4 * nbuf
            for p in range(nbuf):
                gwait(j + p, p)
                pltpu.sync_copy(bufs[p], acc.at[sv.at[j + p]], add=True)
                gfire(j + p + nbuf, p)
            return 0

        lax.fori_loop(0, K // nbuf - 1, body, 0)
        for p in range(nbuf):
            jlast = K - nbuf + p
            gwait(jlast, p)
            pltpu.sync_copy(bufs[p], acc.at[sv.at[jlast]], add=True)
        plsc.subcore_barrier()

        nblk = rows_pt // 64

        def cin(b, p):
            pltpu.async_copy(acc.at[pl.ds(r0 + b * 64, 64)],
                             bufs[p].at[pl.ds(0, 64)], gsems[p])

        def cin_wait(b, p):
            pltpu.make_async_copy(acc.at[pl.ds(r0 + b * 64, 64)],
                                  bufs[p].at[pl.ds(0, 64)], gsems[p]).wait()

        cin(0, 0)
        for b in range(nblk):
            p = b & 1
            blk0 = r0 + b * 64
            cin_wait(b, p)
            if b + 1 < nblk:
                cin(b + 1, 1 - p)

            def srow(r, _, b=b, p=p):
                sval = scb[b * 64 + r]
                for d in range(dh // L):
                    bufs[p][r, pl.ds(d * L, L)] = (
                        bufs[p][r, pl.ds(d * L, L)] * sval)
                return 0

            lax.fori_loop(0, 64, srow, 0)
            if interleave:
                @pl.when(cid == 0)
                def _(p=p, blk0=blk0):
                    pltpu.sync_copy(bufs[p].at[pl.ds(0, 64)],
                                    dst.at[pl.ds(blk0, 64), pl.ds(0, dh)])

                @pl.when(cid == 1)
                def _(p=p, blk0=blk0):
                    pltpu.sync_copy(bufs[p].at[pl.ds(0, 64)],
                                    dst.at[pl.ds(blk0, 64), pl.ds(dh, dh)])
            else:
                pltpu.sync_copy(bufs[p].at[pl.ds(0, 64)],
                                dst.at[pl.ds(cid * rdst + blk0, 64)])

    return stage


def _make_degrees():
    mesh = plsc.VectorSubcoreMesh(
        core_axis_name="c", subcore_axis_name="s",
        num_cores=NC, num_subcores=NS)
    n_pt = NP // NS
    m_pt = MP // NS

    @functools.partial(
        pl.kernel,
        out_type=(jax.ShapeDtypeStruct((NP, L), jnp.float32),
                  jax.ShapeDtypeStruct((MP, L), jnp.float32)),
        mesh=mesh,
        compiler_params=pltpu.CompilerParams(use_tc_tiling_on_sc=False),
        scratch_types=[
            pltpu.VMEM((K, CHUNK), jnp.int32),
            pltpu.VMEM((CHUNK, L), jnp.float32),
            pltpu.VMEM_SHARED((NP, L), jnp.float32),
            pltpu.SemaphoreType.DMA,
        ],
    )
    def degrees(sn, se, zerosN, onesH, outN, outM, sv, ones, acc, dsem):
        cid = lax.axis_index("c")
        sid = lax.axis_index("s")
        pltpu.sync_copy(onesH, ones)

        @pl.when(cid == 0)
        def _():
            pltpu.sync_copy(sn.at[sid], sv)
            pltpu.sync_copy(zerosN.at[pl.ds(sid * n_pt, n_pt)],
                            acc.at[pl.ds(sid * n_pt, n_pt)])

        @pl.when(cid == 1)
        def _():
            pltpu.sync_copy(se.at[sid], sv)
            pltpu.sync_copy(zerosN.at[pl.ds(sid * m_pt, m_pt)],
                            acc.at[pl.ds(sid * m_pt, m_pt)])

        plsc.subcore_barrier()

        def body(j, _):
            pltpu.async_copy(ones, acc.at[sv.at[j]], dsem, add=True)
            return 0

        lax.fori_loop(0, K, body, 0)

        def drain(j, _):
            pltpu.make_async_copy(ones, acc.at[sv.at[0]], dsem).wait()
            return 0

        lax.fori_loop(0, K, drain, 0)
        plsc.subcore_barrier()

        @pl.when(cid == 0)
        def _():
            pltpu.sync_copy(acc.at[pl.ds(sid * n_pt, n_pt)],
                            outN.at[pl.ds(sid * n_pt, n_pt)])

        @pl.when(cid == 1)
        def _():
            pltpu.sync_copy(acc.at[pl.ds(sid * m_pt, m_pt)],
                            outM.at[pl.ds(sid * m_pt, m_pt)])

    return degrees


_make_stage = functools.lru_cache(maxsize=None)(_make_stage)
_make_degrees = functools.lru_cache(maxsize=None)(_make_degrees)


def kernel(X, node_idx, edge_idx, W1, b1, W2, b2):
    node_idx = node_idx.astype(jnp.int32)
    edge_idx = edge_idx.astype(jnp.int32)

    pad = NS * NNZ_PT - NNZ
    spread = jnp.arange(pad, dtype=jnp.int32)
    gn = jnp.concatenate([node_idx, spread % N]).reshape(NS, K, CHUNK)
    ge = jnp.concatenate([edge_idx, spread % M]).reshape(NS, K, CHUNK)
    sn = jnp.concatenate([node_idx, N + spread % (NP - N)]).reshape(NS, K, CHUNK)
    se = jnp.concatenate([edge_idx, M + spread % (MP - M)]).reshape(NS, K, CHUNK)
    gn_off = jnp.stack([gn, gn + NP]).reshape(NC * NS, K, CHUNK)
    ge_off = jnp.stack([ge, ge + MP]).reshape(NC * NS, K, CHUNK)

    zerosN = jnp.zeros((NP, L), jnp.float32)
    onesH = jnp.ones((CHUNK, L), jnp.float32)
    cNw, cMw = _make_degrees()(sn, se, zerosN, onesH)

    Xp = jnp.pad(X, ((0, NP - N), (0, 0)))
    h, dvw, dew = _tc_mm1(Xp, W1, b1.reshape(1, DHID), cNw, cMw)

    e1 = _make_stage(NP, MP, 64)(h.reshape(2 * NP, 64), gn_off, se, dew)
    n1 = _make_stage(MP, NP, 64)(e1, ge_off, sn, dvw)

    W2p = jnp.pad(W2, ((0, 0), (0, 64 - NCLS)))
    b2r = jnp.pad(b2, (0, 64 - NCLS)).reshape(1, 64)
    h2 = _tc_mm2(n1.reshape(2, NP, 64), cNw[:, :1], W2p, b2r)

    e2 = _make_stage(NP, MP, 32)(h2.reshape(2 * NP, 32), gn_off, se, dew)
    out = _make_stage(MP, NP, 32, interleave=True)(e2, ge_off, sn, dvw)
    return out[:N, :NCLS]
